# baseline (speedup 1.0000x reference)
import functools

import jax
import jax.numpy as jnp
from jax import lax
from jax.experimental import pallas as pl
from jax.experimental.pallas import tpu as pltpu
from jax.experimental.pallas import tpu_sc as plsc


def _enc_body(x_ref, w_ref, o_ref):
    xb = x_ref[...]
    w = w_ref[...]
    h = xb[:, 0:1] * w[0:1, :] + xb[:, 1:2] * w[1:2, :]
    o_ref[...] = jnp.concatenate([jnp.sin(h), jnp.cos(h)], axis=-1)


def _spa_encode(x2, w, rb):
    rows, _ = x2.shape
    half = w.shape[1]
    grid = rows // rb
    return pl.pallas_call(
        _enc_body,
        grid=(grid,),
        in_specs=[
            pl.BlockSpec((rb, x2.shape[1]), lambda i: (i, 0)),
            pl.BlockSpec(w.shape, lambda i: (0, 0)),
        ],
        out_specs=pl.BlockSpec((rb, 2 * half), lambda i: (i, 0)),
        out_shape=jax.ShapeDtypeStruct((rows, 2 * half), jnp.float32),
    )(x2, w)


def _make_time_encode(B, F, V, D):
    info = plsc.get_sparse_core_info()
    NC, NS, L = info.num_cores, info.num_subcores, info.num_lanes
    NW = NC * NS
    bpw = B // NW
    npw = bpw * F
    nchunk = npw // 128
    mesh = plsc.VectorSubcoreMesh(core_axis_name="c", subcore_axis_name="s")

    @functools.partial(
        pl.kernel,
        mesh=mesh,
        out_type=jax.ShapeDtypeStruct((B, D), jnp.float32),
        scratch_types=[
            pltpu.VMEM((npw,), jnp.int32),
            pltpu.VMEM((nchunk, 128), jnp.int32),
            pltpu.VMEM((npw, D), jnp.float32),
            pltpu.VMEM((bpw, D), jnp.float32),
            pltpu.SemaphoreType.DMA,
        ],
    )
    def k(t_hbm, table_hbm, out_hbm, t_v, idx_v, rows_v, out_v, sem):
        wid = lax.axis_index("s") * NC + lax.axis_index("c")
        pltpu.sync_copy(t_hbm.at[pl.ds(wid * npw, npw)], t_v)
        off = (lax.iota(jnp.int32, L) % F) * V
        for kk in range(npw // L):
            c, j = kk // (128 // L), kk % (128 // L)
            idx_v[c, pl.ds(j * L, L)] = t_v[pl.ds(kk * L, L)] + off
        copies = [
            pltpu.async_copy(
                table_hbm.at[idx_v.at[c]],
                rows_v.at[pl.ds(c * 128, 128)],
                sem,
            )
            for c in range(nchunk)
        ]
        for cp in copies:
            cp.wait()

        def row_body(r, carry):
            for ch in range(D // L):
                sl = pl.ds(ch * L, L)
                acc = rows_v[F * r, sl]
                for i in range(1, F):
                    acc = acc + rows_v[F * r + i, sl]
                out_v[r, sl] = acc
            return carry

        lax.fori_loop(0, bpw, row_body, 0)
        pltpu.sync_copy(out_v, out_hbm.at[pl.ds(wid * bpw, bpw)])

    return k


def kernel(x, t, W_spa, embed_matrix):
    B, Lseq, _ = x.shape
    F, V, D = embed_matrix.shape
    x2 = x.astype(jnp.float32).reshape(B * Lseq, x.shape[-1])
    x_enc = _spa_encode(x2, W_spa.astype(jnp.float32), rb=2048)
    x_enc = x_enc.reshape(B, Lseq, 2 * W_spa.shape[1])

    t_flat = t.astype(jnp.int32).reshape(B * F)
    table = embed_matrix.astype(jnp.float32).reshape(F * V, D)
    t_enc = _make_time_encode(B, F, V, D)(t_flat, table)
    return (x_enc, t_enc[:, None, :])


# speedup vs baseline: 1.4965x; 1.4965x over previous
import functools

import jax
import jax.numpy as jnp
from jax import lax
from jax.experimental import pallas as pl
from jax.experimental.pallas import tpu as pltpu
from jax.experimental.pallas import tpu_sc as plsc


_INV_2PI = 0.15915494309189535
_2PI_HI = 6.28318548202514648437500
_2PI_LO = -1.74845552749576962e-7

_C1 = 0.9999845938223363
_C2 = -0.16663259442014317
_C3 = 0.008312388576924809
_C4 = -0.00019316274764465124
_C5 = 2.1732595674173183e-06


def _sin_phased(h):
    nf = jnp.floor(h * _INV_2PI + 0.5)
    r = (h - nf * _2PI_HI) - nf * _2PI_LO
    r2 = r * r
    return r * (_C1 + r2 * (_C2 + r2 * (_C3 + r2 * (_C4 + r2 * _C5))))


def _enc_body(x_ref, w_ref, o_ref):
    h = jnp.dot(x_ref[...], w_ref[...], preferred_element_type=jnp.float32)
    o_ref[...] = _sin_phased(h)


def _spa_encode(x2, w, rb):
    rows = x2.shape[0]
    half = w.shape[1]
    x3 = jnp.concatenate([x2, jnp.ones((rows, 1), jnp.float32)], axis=1)
    shift = jnp.concatenate([jnp.zeros((1, half), jnp.float32),
                             jnp.full((1, half), 0.5 * jnp.pi, jnp.float32)],
                            axis=1)
    w3 = jnp.concatenate([jnp.concatenate([w, w], axis=1), shift], axis=0)
    grid = rows // rb
    return pl.pallas_call(
        _enc_body,
        grid=(grid,),
        in_specs=[
            pl.BlockSpec((rb, 3), lambda i: (i, 0)),
            pl.BlockSpec((3, 2 * half), lambda i: (0, 0)),
        ],
        out_specs=pl.BlockSpec((rb, 2 * half), lambda i: (i, 0)),
        out_shape=jax.ShapeDtypeStruct((rows, 2 * half), jnp.float32),
    )(x3, w3)


def _make_time_encode(B, F, V, D):
    info = plsc.get_sparse_core_info()
    NC, NS, L = info.num_cores, info.num_subcores, info.num_lanes
    NW = NC * NS
    bpw = B // NW
    npw = bpw * F
    nchunk = npw // 128
    mesh = plsc.VectorSubcoreMesh(core_axis_name="c", subcore_axis_name="s")

    @functools.partial(
        pl.kernel,
        mesh=mesh,
        out_type=jax.ShapeDtypeStruct((B, D), jnp.float32),
        scratch_types=[
            pltpu.VMEM((npw,), jnp.int32),
            pltpu.VMEM((nchunk, 128), jnp.int32),
            pltpu.VMEM((npw, D), jnp.float32),
            pltpu.VMEM((bpw, D), jnp.float32),
            pltpu.SemaphoreType.DMA,
        ],
    )
    def k(t_hbm, table_hbm, out_hbm, t_v, idx_v, rows_v, out_v, sem):
        wid = lax.axis_index("s") * NC + lax.axis_index("c")
        pltpu.sync_copy(t_hbm.at[pl.ds(wid * npw, npw)], t_v)
        off = (lax.iota(jnp.int32, L) % F) * V
        for kk in range(npw // L):
            c, j = kk // (128 // L), kk % (128 // L)
            idx_v[c, pl.ds(j * L, L)] = t_v[pl.ds(kk * L, L)] + off
        copies = [
            pltpu.async_copy(
                table_hbm.at[idx_v.at[c]],
                rows_v.at[pl.ds(c * 128, 128)],
                sem,
            )
            for c in range(nchunk)
        ]
        for cp in copies:
            cp.wait()

        def row_body(r, carry):
            for ch in range(D // L):
                sl = pl.ds(ch * L, L)
                acc = rows_v[F * r, sl]
                for i in range(1, F):
                    acc = acc + rows_v[F * r + i, sl]
                out_v[r, sl] = acc
            return carry

        lax.fori_loop(0, bpw, row_body, 0)
        pltpu.sync_copy(out_v, out_hbm.at[pl.ds(wid * bpw, bpw)])

    return k


def kernel(x, t, W_spa, embed_matrix):
    B, Lseq, _ = x.shape
    F, V, D = embed_matrix.shape
    x2 = x.astype(jnp.float32).reshape(B * Lseq, x.shape[-1])
    x_enc = _spa_encode(x2, W_spa.astype(jnp.float32), rb=2048)
    x_enc = x_enc.reshape(B, Lseq, 2 * W_spa.shape[1])

    t_flat = t.astype(jnp.int32).reshape(B * F)
    table = embed_matrix.astype(jnp.float32).reshape(F * V, D)
    t_enc = _make_time_encode(B, F, V, D)(t_flat, table)
    return (x_enc, t_enc[:, None, :])


# speedup vs baseline: 5.8087x; 3.8814x over previous
import functools

import jax
import jax.numpy as jnp
from jax import lax
from jax.experimental import pallas as pl
from jax.experimental.pallas import tpu as pltpu
from jax.experimental.pallas import tpu_sc as plsc


_INV_2PI = 0.15915494309189535
_2PI_HI = 6.28318548202514648437500
_2PI_LO = -1.74845552749576962e-7

_C1 = 0.9999845938223363
_C2 = -0.16663259442014317
_C3 = 0.008312388576924809
_C4 = -0.00019316274764465124
_C5 = 2.1732595674173183e-06


def _sin_phased(h):
    nf = jnp.floor(h * _INV_2PI + 0.5)
    r = (h - nf * _2PI_HI) - nf * _2PI_LO
    r2 = r * r
    return r * (_C1 + r2 * (_C2 + r2 * (_C3 + r2 * (_C4 + r2 * _C5))))


def _enc_body(xt_ref, w_ref, o_ref):
    h = lax.dot_general(xt_ref[...], w_ref[...],
                        dimension_numbers=(((0,), (0,)), ((), ())),
                        preferred_element_type=jnp.float32)
    o_ref[...] = _sin_phased(h)


def _spa_encode(xt3, w3, rb):
    rows = xt3.shape[1]
    grid = rows // rb
    return pl.pallas_call(
        _enc_body,
        grid=(grid,),
        in_specs=[
            pl.BlockSpec((3, rb), lambda i: (0, i)),
            pl.BlockSpec((3, 128), lambda i: (0, 0)),
        ],
        out_specs=pl.BlockSpec((rb, 128), lambda i: (i, 0)),
        out_shape=jax.ShapeDtypeStruct((rows, 128), jnp.float32),
    )(xt3, w3)


def _make_time_encode(B, F, V, D):
    info = plsc.get_sparse_core_info()
    NC, NS, L = info.num_cores, info.num_subcores, info.num_lanes
    NW = NC * NS
    bpw = B // NW
    mesh = plsc.VectorSubcoreMesh(core_axis_name="c", subcore_axis_name="s")

    @functools.partial(
        pl.kernel,
        mesh=mesh,
        out_type=jax.ShapeDtypeStruct((B, D), jnp.float32),
        scratch_types=[
            pltpu.VMEM((F, bpw), jnp.int32),
            pltpu.VMEM((F, bpw), jnp.int32),
            pltpu.VMEM((F * bpw, D), jnp.float32),
            pltpu.VMEM((bpw, D), jnp.float32),
            pltpu.SemaphoreType.DMA,
        ],
    )
    def k(t_hbm, table_hbm, out_hbm, t_v, idx_v, rows_v, out_v, sem):
        wid = lax.axis_index("s") * NC + lax.axis_index("c")
        base = wid * bpw
        for c in range(F):
            pltpu.sync_copy(t_hbm.at[pl.ds(c * B + base, bpw)], t_v.at[c])
        for c in range(F):
            for j in range(bpw // L):
                sl = pl.ds(j * L, L)
                idx_v[c, sl] = t_v[c, sl] * F + c
        copies = [
            pltpu.async_copy(
                table_hbm.at[idx_v.at[c]],
                rows_v.at[pl.ds(c * bpw, bpw)],
                sem,
            )
            for c in range(F)
        ]
        for cp in copies:
            cp.wait()

        def row_body(r, carry):
            for ch in range(D // L):
                sl = pl.ds(ch * L, L)
                acc = rows_v[r, sl]
                for c in range(1, F):
                    acc = acc + rows_v[c * bpw + r, sl]
                out_v[r, sl] = acc
            return carry

        lax.fori_loop(0, bpw, row_body, 0)
        pltpu.sync_copy(out_v, out_hbm.at[pl.ds(base, bpw)])

    return k


def kernel(x, t, W_spa, embed_matrix):
    B, Lseq, C = x.shape
    F, V, D = embed_matrix.shape
    half = W_spa.shape[1]

    xt = x.astype(jnp.float32).transpose(2, 1, 0).reshape(C, Lseq * B)
    ones = jnp.ones((1, Lseq * B), jnp.float32)
    xt3 = jnp.concatenate([xt, ones], axis=0)
    wd = jnp.concatenate([W_spa.astype(jnp.float32)] * 2, axis=1)
    shift = jnp.concatenate([jnp.zeros((1, half), jnp.float32),
                             jnp.full((1, half), 0.5 * jnp.pi, jnp.float32)],
                            axis=1)
    w3 = jnp.concatenate([wd, shift], axis=0)
    enc = _spa_encode(xt3, w3, rb=2048)
    x_enc = enc.reshape(Lseq, B, 2 * half).transpose(1, 0, 2)

    t_flat = t.astype(jnp.int32).transpose(1, 0).reshape(F * B)
    table = embed_matrix.astype(jnp.float32).transpose(1, 0, 2).reshape(V * F, D)
    t_enc = _make_time_encode(B, F, V, D)(t_flat, table)
    return (x_enc, t_enc[:, None, :])
